# baseline (speedup 1.0000x reference)
import functools
import jax
import jax.numpy as jnp
from jax import lax
from jax.experimental import pallas as pl
from jax.experimental.pallas import tpu as pltpu
from jax.experimental.pallas import tpu_sc as plsc

N = 10000
E = 320000
D = 128
DA = 144
NPAD = N + 8
NC, NS = 2, 16
NW = NC * NS
K = 64
EPW = 10240
E_PAD = EPW * NW
CHUNKS = EPW // K
NACC = 10112
ROWS_PER_TILE = NACC // NS


def _sc_aggregate(x_aug, src, dst, zrows):
    mesh = plsc.VectorSubcoreMesh(core_axis_name="c", subcore_axis_name="s")

    @functools.partial(
        pl.kernel,
        out_type=jax.ShapeDtypeStruct((NC, NACC, DA), jnp.float32),
        mesh=mesh,
        scratch_types=[
            pltpu.VMEM((CHUNKS, K), jnp.int32),
            pltpu.VMEM((CHUNKS, K), jnp.int32),
            pltpu.VMEM((K, DA), jnp.float32),
            pltpu.VMEM((K, DA), jnp.float32),
            pltpu.VMEM_SHARED((NACC, DA), jnp.float32),
            pltpu.SemaphoreType.DMA,
            pltpu.SemaphoreType.DMA,
            pltpu.SemaphoreType.DMA,
            pltpu.SemaphoreType.DMA,
        ],
        compiler_params=pltpu.CompilerParams(use_tc_tiling_on_sc=False),
    )
    def body(x_ref, src_ref, dst_ref, z_ref, out_ref,
             sidx, didx, rows0, rows1, acc,
             gsem0, gsem1, ssem0, ssem1):
        c = lax.axis_index("c")
        s = lax.axis_index("s")
        w = s * NC + c

        rows = (rows0, rows1)
        gsem = (gsem0, gsem1)
        ssem = (ssem0, ssem1)

        def g_start(i, b):
            for q in range(4):
                pltpu.make_async_copy(
                    x_ref.at[sidx.at[i, pl.ds(q * 16, 16)]],
                    rows[b].at[pl.ds(q * 16, 16)],
                    gsem[b]).start()

        def g_wait(b):
            pltpu.make_async_copy(x_ref.at[sidx.at[0]], rows[b], gsem[b]).wait()

        def s_start(i, b):
            pltpu.make_async_copy(rows[b], acc.at[didx.at[i]], ssem[b]).start(add=True)

        def s_wait(b):
            pltpu.make_async_copy(rows[b], acc.at[didx.at[0]], ssem[b]).wait()

        pltpu.make_async_copy(src_ref.at[w], sidx, gsem0).start()
        pltpu.make_async_copy(dst_ref.at[w], didx, gsem1).start()
        pltpu.make_async_copy(
            z_ref, acc.at[pl.ds(s * ROWS_PER_TILE, ROWS_PER_TILE)], ssem0).start()

        pltpu.make_async_copy(src_ref.at[w], sidx, gsem0).wait()
        pltpu.make_async_copy(dst_ref.at[w], didx, gsem1).wait()
        pltpu.make_async_copy(
            z_ref, acc.at[pl.ds(s * ROWS_PER_TILE, ROWS_PER_TILE)], ssem0).wait()
        plsc.subcore_barrier()

        g_start(0, 0)
        g_wait(0)
        s_start(0, 0)
        g_start(1, 1)

        def step(t, carry):
            i1 = 2 * t + 1
            g_wait(1)
            s_start(i1, 1)
            s_wait(0)
            g_start(i1 + 1, 0)
            i2 = 2 * t + 2
            g_wait(0)
            s_start(i2, 0)
            s_wait(1)
            g_start(i2 + 1, 1)
            return carry
        lax.fori_loop(0, (CHUNKS - 2) // 2, step, 0)

        g_wait(1)
        s_start(CHUNKS - 1, 1)
        s_wait(0)
        s_wait(1)
        plsc.subcore_barrier()

        pltpu.sync_copy(
            acc.at[pl.ds(s * ROWS_PER_TILE, ROWS_PER_TILE)],
            out_ref.at[c, pl.ds(s * ROWS_PER_TILE, ROWS_PER_TILE)],
        )

    return body(x_aug, src, dst, zrows)


R = 400


def _tc_finish(x, psum, W_self, W_neigh, bias, gamma, beta):
    def body(x_ref, p_ref, ws_ref, wn_ref, b_ref, g_ref, be_ref, o_ref):
        p = p_ref[...]
        ssum = p[0] + p[1]
        agg = ssum[:, :D]
        deg = jnp.maximum(ssum[:, D], 1.0)
        neigh = agg / deg[:, None]
        xv = x_ref[...]
        dn = (((1,), (1,)), ((), ()))
        out = (lax.dot_general(xv, ws_ref[...], dn, preferred_element_type=jnp.float32)
               + lax.dot_general(neigh, wn_ref[...], dn, preferred_element_type=jnp.float32)
               + b_ref[...])
        out = jnp.maximum(out, 0.0)
        mu = jnp.mean(out, axis=-1, keepdims=True)
        var = jnp.mean((out - mu) ** 2, axis=-1, keepdims=True)
        o_ref[...] = ((out - mu) * lax.rsqrt(var + 1e-5)) * g_ref[...] + be_ref[...]

    return pl.pallas_call(
        body,
        grid=(N // R,),
        in_specs=[
            pl.BlockSpec((R, D), lambda i: (i, 0)),
            pl.BlockSpec((NC, R, DA), lambda i: (0, i, 0)),
            pl.BlockSpec((D, D), lambda i: (0, 0)),
            pl.BlockSpec((D, D), lambda i: (0, 0)),
            pl.BlockSpec((1, D), lambda i: (0, 0)),
            pl.BlockSpec((1, D), lambda i: (0, 0)),
            pl.BlockSpec((1, D), lambda i: (0, 0)),
        ],
        out_specs=pl.BlockSpec((R, D), lambda i: (i, 0)),
        out_shape=jax.ShapeDtypeStruct((N, D), jnp.float32),
    )(x, psum, W_self, W_neigh, bias, gamma, beta)


def kernel(x, edge_index, W_self, W_neigh, bias, ln_gamma, ln_beta):
    src = edge_index[0].astype(jnp.int32)
    dst = edge_index[1].astype(jnp.int32)
    pad = E_PAD - E
    src = jnp.concatenate([src, jnp.full((pad,), N, jnp.int32)]).reshape(NW, CHUNKS, K)
    dst = jnp.concatenate([dst, jnp.zeros((pad,), jnp.int32)]).reshape(NW, CHUNKS, K)
    x_aug = jnp.zeros((NPAD, DA), jnp.float32)
    x_aug = x_aug.at[:N, :D].set(x).at[:N, D].set(1.0)
    zrows = jnp.zeros((ROWS_PER_TILE, DA), jnp.float32)
    psum = _sc_aggregate(x_aug, src, dst, zrows)
    return _tc_finish(
        x, psum, W_self, W_neigh,
        bias.reshape(1, D), ln_gamma.reshape(1, D), ln_beta.reshape(1, D),
    )


# speedup vs baseline: 1.0006x; 1.0006x over previous
import functools
import jax
import jax.numpy as jnp
from jax import lax
from jax.experimental import pallas as pl
from jax.experimental.pallas import tpu as pltpu
from jax.experimental.pallas import tpu_sc as plsc

N = 10000
E = 320000
D = 128
DA = 144
NPAD = N + 8
NC, NS = 2, 16
NW = NC * NS
K = 64
EPW = 10240
E_PAD = EPW * NW
CHUNKS = EPW // K
NACC = 10112
ROWS_PER_TILE = NACC // NS


def _sc_aggregate(x_aug, src, dst, zrows):
    mesh = plsc.VectorSubcoreMesh(core_axis_name="c", subcore_axis_name="s")

    @functools.partial(
        pl.kernel,
        out_type=jax.ShapeDtypeStruct((NC, NACC, DA), jnp.float32),
        mesh=mesh,
        scratch_types=[
            pltpu.VMEM((CHUNKS, K), jnp.int32),
            pltpu.VMEM((CHUNKS, K), jnp.int32),
            pltpu.VMEM((K, DA), jnp.float32),
            pltpu.VMEM((K, DA), jnp.float32),
            pltpu.VMEM_SHARED((NACC, DA), jnp.float32),
            pltpu.SemaphoreType.DMA,
            pltpu.SemaphoreType.DMA,
            pltpu.SemaphoreType.DMA,
            pltpu.SemaphoreType.DMA,
        ],
        compiler_params=pltpu.CompilerParams(use_tc_tiling_on_sc=False),
    )
    def body(x_ref, src_ref, dst_ref, z_ref, out_ref,
             sidx, didx, rows0, rows1, acc,
             gsem0, gsem1, ssem0, ssem1):
        c = lax.axis_index("c")
        s = lax.axis_index("s")
        w = s * NC + c

        rows = (rows0, rows1)
        gsem = (gsem0, gsem1)
        ssem = (ssem0, ssem1)

        def g_start(i, b):
            for q in range(4):
                pltpu.make_async_copy(
                    x_ref.at[sidx.at[i, pl.ds(q * 16, 16)]],
                    rows[b].at[pl.ds(q * 16, 16)],
                    gsem[b]).start()

        def g_wait(b):
            pltpu.make_async_copy(x_ref.at[sidx.at[0]], rows[b], gsem[b]).wait()

        def s_start(i, b):
            pltpu.make_async_copy(rows[b], acc.at[didx.at[i]], ssem[b]).start(add=True)

        def s_wait(b):
            pltpu.make_async_copy(rows[b], acc.at[didx.at[0]], ssem[b]).wait()

        with jax.named_scope("stage_zero"):
            pltpu.make_async_copy(src_ref.at[w], sidx, gsem0).start()
            pltpu.make_async_copy(dst_ref.at[w], didx, gsem1).start()
            pltpu.make_async_copy(
                z_ref, acc.at[pl.ds(s * ROWS_PER_TILE, ROWS_PER_TILE)], ssem0).start()

            pltpu.make_async_copy(src_ref.at[w], sidx, gsem0).wait()
            pltpu.make_async_copy(dst_ref.at[w], didx, gsem1).wait()
            pltpu.make_async_copy(
                z_ref, acc.at[pl.ds(s * ROWS_PER_TILE, ROWS_PER_TILE)], ssem0).wait()
            plsc.subcore_barrier()

        with jax.named_scope("edges"):
            g_start(0, 0)
            g_wait(0)
            s_start(0, 0)
            g_start(1, 1)

            def step(t, carry):
                i1 = 2 * t + 1
                g_wait(1)
                s_start(i1, 1)
                s_wait(0)
                g_start(i1 + 1, 0)
                i2 = 2 * t + 2
                g_wait(0)
                s_start(i2, 0)
                s_wait(1)
                g_start(i2 + 1, 1)
                return carry
            lax.fori_loop(0, (CHUNKS - 2) // 2, step, 0)

            g_wait(1)
            s_start(CHUNKS - 1, 1)
            s_wait(0)
            s_wait(1)
            plsc.subcore_barrier()

        with jax.named_scope("dump"):
            pltpu.sync_copy(
                acc.at[pl.ds(s * ROWS_PER_TILE, ROWS_PER_TILE)],
                out_ref.at[c, pl.ds(s * ROWS_PER_TILE, ROWS_PER_TILE)],
            )

    return body(x_aug, src, dst, zrows)


R = 400


def _tc_finish(x, psum, W_self, W_neigh, bias, gamma, beta):
    def body(x_ref, p_ref, ws_ref, wn_ref, b_ref, g_ref, be_ref, o_ref):
        p = p_ref[...]
        ssum = p[0] + p[1]
        agg = ssum[:, :D]
        deg = jnp.maximum(ssum[:, D], 1.0)
        neigh = agg / deg[:, None]
        xv = x_ref[...]
        dn = (((1,), (1,)), ((), ()))
        out = (lax.dot_general(xv, ws_ref[...], dn, preferred_element_type=jnp.float32)
               + lax.dot_general(neigh, wn_ref[...], dn, preferred_element_type=jnp.float32)
               + b_ref[...])
        out = jnp.maximum(out, 0.0)
        mu = jnp.mean(out, axis=-1, keepdims=True)
        var = jnp.mean((out - mu) ** 2, axis=-1, keepdims=True)
        o_ref[...] = ((out - mu) * lax.rsqrt(var + 1e-5)) * g_ref[...] + be_ref[...]

    return pl.pallas_call(
        body,
        grid=(N // R,),
        in_specs=[
            pl.BlockSpec((R, D), lambda i: (i, 0)),
            pl.BlockSpec((NC, R, DA), lambda i: (0, i, 0)),
            pl.BlockSpec((D, D), lambda i: (0, 0)),
            pl.BlockSpec((D, D), lambda i: (0, 0)),
            pl.BlockSpec((1, D), lambda i: (0, 0)),
            pl.BlockSpec((1, D), lambda i: (0, 0)),
            pl.BlockSpec((1, D), lambda i: (0, 0)),
        ],
        out_specs=pl.BlockSpec((R, D), lambda i: (i, 0)),
        out_shape=jax.ShapeDtypeStruct((N, D), jnp.float32),
    )(x, psum, W_self, W_neigh, bias, gamma, beta)


def kernel(x, edge_index, W_self, W_neigh, bias, ln_gamma, ln_beta):
    src = edge_index[0].astype(jnp.int32)
    dst = edge_index[1].astype(jnp.int32)
    pad = E_PAD - E
    src = jnp.concatenate([src, jnp.full((pad,), N, jnp.int32)]).reshape(NW, CHUNKS, K)
    dst = jnp.concatenate([dst, jnp.zeros((pad,), jnp.int32)]).reshape(NW, CHUNKS, K)
    x_aug = jnp.zeros((NPAD, DA), jnp.float32)
    x_aug = x_aug.at[:N, :D].set(x).at[:N, D].set(1.0)
    zrows = jnp.zeros((ROWS_PER_TILE, DA), jnp.float32)
    psum = _sc_aggregate(x_aug, src, dst, zrows)
    return _tc_finish(
        x, psum, W_self, W_neigh,
        bias.reshape(1, D), ln_gamma.reshape(1, D), ln_beta.reshape(1, D),
    )


# speedup vs baseline: 1.2958x; 1.2950x over previous
import functools
import jax
import jax.numpy as jnp
from jax import lax
from jax.experimental import pallas as pl
from jax.experimental.pallas import tpu as pltpu
from jax.experimental.pallas import tpu_sc as plsc

N = 10000
E = 320000
D = 128
DA = 144
NPAD = N + 8
NC, NS = 2, 16
NW = NC * NS
K = 40
CH_F = 338
CH_S = 162
CHUNKS = CH_F
NACC = 10112
ROWS_PER_TILE = NACC // NS


def _sc_aggregate(x_aug, src, dst, zrows):
    mesh = plsc.VectorSubcoreMesh(core_axis_name="c", subcore_axis_name="s")

    @functools.partial(
        pl.kernel,
        out_type=jax.ShapeDtypeStruct((NC, NACC, DA), jnp.float32),
        mesh=mesh,
        scratch_types=[
            pltpu.VMEM((CHUNKS, K), jnp.int32),
            pltpu.VMEM((CHUNKS, K), jnp.int32),
            pltpu.VMEM((K, DA), jnp.float32),
            pltpu.VMEM((K, DA), jnp.float32),
            pltpu.VMEM_SHARED((NACC, DA), jnp.float32),
            pltpu.SemaphoreType.DMA,
            pltpu.SemaphoreType.DMA,
            pltpu.SemaphoreType.DMA,
            pltpu.SemaphoreType.DMA,
        ],
        compiler_params=pltpu.CompilerParams(use_tc_tiling_on_sc=False),
    )
    def body(x_ref, src_ref, dst_ref, z_ref, out_ref,
             sidx, didx, rows0, rows1, acc,
             gsem0, gsem1, ssem0, ssem1):
        c = lax.axis_index("c")
        s = lax.axis_index("s")
        w = s * NC + c

        rows = (rows0, rows1)
        gsem = (gsem0, gsem1)
        ssem = (ssem0, ssem1)

        def g_start(i, b):
            pltpu.make_async_copy(x_ref.at[sidx.at[i]], rows[b], gsem[b]).start()

        def g_wait(b):
            pltpu.make_async_copy(x_ref.at[sidx.at[0]], rows[b], gsem[b]).wait()

        def s_start(i, b):
            pltpu.make_async_copy(rows[b], acc.at[didx.at[i]], ssem[b]).start(add=True)

        def s_wait(b):
            pltpu.make_async_copy(rows[b], acc.at[didx.at[0]], ssem[b]).wait()

        with jax.named_scope("stage_zero"):
            pltpu.make_async_copy(src_ref.at[w], sidx, gsem0).start()
            pltpu.make_async_copy(dst_ref.at[w], didx, gsem1).start()
            pltpu.make_async_copy(
                z_ref, acc.at[pl.ds(s * ROWS_PER_TILE, ROWS_PER_TILE)], ssem0).start()

            pltpu.make_async_copy(src_ref.at[w], sidx, gsem0).wait()
            pltpu.make_async_copy(dst_ref.at[w], didx, gsem1).wait()
            pltpu.make_async_copy(
                z_ref, acc.at[pl.ds(s * ROWS_PER_TILE, ROWS_PER_TILE)], ssem0).wait()
            plsc.subcore_barrier()

        with jax.named_scope("edges"):
            g_start(0, 0)
            g_wait(0)
            s_start(0, 0)
            g_start(1, 1)

            def step(t, carry):
                i1 = 2 * t + 1
                g_wait(1)
                s_start(i1, 1)
                s_wait(0)
                g_start(i1 + 1, 0)
                i2 = 2 * t + 2
                g_wait(0)
                s_start(i2, 0)
                s_wait(1)
                g_start(i2 + 1, 1)
                return carry
            my_chunks = jnp.where(c == 0, CH_F, CH_S)
            lax.fori_loop(0, (my_chunks - 2) // 2, step, 0)

            g_wait(1)
            s_start(my_chunks - 1, 1)
            s_wait(0)
            s_wait(1)
            plsc.subcore_barrier()

        with jax.named_scope("dump"):
            pltpu.sync_copy(
                acc.at[pl.ds(s * ROWS_PER_TILE, ROWS_PER_TILE)],
                out_ref.at[c, pl.ds(s * ROWS_PER_TILE, ROWS_PER_TILE)],
            )

    return body(x_aug, src, dst, zrows)


R = 400


def _tc_finish(x, psum, W_self, W_neigh, bias, gamma, beta):
    def body(x_ref, p_ref, ws_ref, wn_ref, b_ref, g_ref, be_ref, o_ref):
        p = p_ref[...]
        ssum = p[0] + p[1]
        agg = ssum[:, :D]
        deg = jnp.maximum(ssum[:, D], 1.0)
        neigh = agg / deg[:, None]
        xv = x_ref[...]
        dn = (((1,), (1,)), ((), ()))
        out = (lax.dot_general(xv, ws_ref[...], dn, preferred_element_type=jnp.float32)
               + lax.dot_general(neigh, wn_ref[...], dn, preferred_element_type=jnp.float32)
               + b_ref[...])
        out = jnp.maximum(out, 0.0)
        mu = jnp.mean(out, axis=-1, keepdims=True)
        var = jnp.mean((out - mu) ** 2, axis=-1, keepdims=True)
        o_ref[...] = ((out - mu) * lax.rsqrt(var + 1e-5)) * g_ref[...] + be_ref[...]

    return pl.pallas_call(
        body,
        grid=(N // R,),
        in_specs=[
            pl.BlockSpec((R, D), lambda i: (i, 0)),
            pl.BlockSpec((NC, R, DA), lambda i: (0, i, 0)),
            pl.BlockSpec((D, D), lambda i: (0, 0)),
            pl.BlockSpec((D, D), lambda i: (0, 0)),
            pl.BlockSpec((1, D), lambda i: (0, 0)),
            pl.BlockSpec((1, D), lambda i: (0, 0)),
            pl.BlockSpec((1, D), lambda i: (0, 0)),
        ],
        out_specs=pl.BlockSpec((R, D), lambda i: (i, 0)),
        out_shape=jax.ShapeDtypeStruct((N, D), jnp.float32),
    )(x, psum, W_self, W_neigh, bias, gamma, beta)


def _split_edges(a):
    ef = NS * CH_F * K
    fast = a[:ef].reshape(NS, CH_F, K)
    slow = a[ef:].reshape(NS, CH_S, K)
    slow = jnp.pad(slow, ((0, 0), (0, CH_F - CH_S), (0, 0)))
    return jnp.stack([fast, slow], axis=1).reshape(NW, CHUNKS, K)


def kernel(x, edge_index, W_self, W_neigh, bias, ln_gamma, ln_beta):
    src = _split_edges(edge_index[0].astype(jnp.int32))
    dst = _split_edges(edge_index[1].astype(jnp.int32))
    x_aug = jnp.zeros((NPAD, DA), jnp.float32)
    x_aug = x_aug.at[:N, :D].set(x).at[:N, D].set(1.0)
    zrows = jnp.zeros((ROWS_PER_TILE, DA), jnp.float32)
    psum = _sc_aggregate(x_aug, src, dst, zrows)
    return _tc_finish(
        x, psum, W_self, W_neigh,
        bias.reshape(1, D), ln_gamma.reshape(1, D), ln_beta.reshape(1, D),
    )


# speedup vs baseline: 1.5890x; 1.2263x over previous
import functools
import jax
import jax.numpy as jnp
from jax import lax
from jax.experimental import pallas as pl
from jax.experimental.pallas import tpu as pltpu
from jax.experimental.pallas import tpu_sc as plsc

N = 10000
E = 320000
D = 128
DA = 144
NPAD = N + 8
NC, NS = 2, 16
NW = NC * NS
K = 40
CH_F = 250
CH_S = 250
CHUNKS = CH_F
NACC = 10112
ROWS_PER_TILE = NACC // NS


def _sc_aggregate(x_aug, src, dst, zrows):
    mesh = plsc.VectorSubcoreMesh(core_axis_name="c", subcore_axis_name="s")

    @functools.partial(
        pl.kernel,
        out_type=jax.ShapeDtypeStruct((NC, NACC, DA), jnp.float32),
        mesh=mesh,
        scratch_types=[
            pltpu.VMEM((CHUNKS, K), jnp.int32),
            pltpu.VMEM((CHUNKS, K), jnp.int32),
            pltpu.VMEM((K, DA), jnp.float32),
            pltpu.VMEM((K, DA), jnp.float32),
            pltpu.VMEM_SHARED((NACC, DA), jnp.float32),
            pltpu.SemaphoreType.DMA,
            pltpu.SemaphoreType.DMA,
            pltpu.SemaphoreType.DMA,
            pltpu.SemaphoreType.DMA,
        ],
        compiler_params=pltpu.CompilerParams(use_tc_tiling_on_sc=False),
    )
    def body(x_ref, src_ref, dst_ref, z_ref, out_ref,
             sidx, didx, rows0, rows1, acc,
             gsem0, gsem1, ssem0, ssem1):
        c = lax.axis_index("c")
        s = lax.axis_index("s")
        w = s * NC + c

        rows = (rows0, rows1)
        gsem = (gsem0, gsem1)
        ssem = (ssem0, ssem1)

        def g_start(i, b):
            pltpu.make_async_copy(x_ref.at[sidx.at[i]], rows[b], gsem[b]).start()

        def g_wait(b):
            pltpu.make_async_copy(x_ref.at[sidx.at[0]], rows[b], gsem[b]).wait()

        def s_start(i, b):
            pltpu.make_async_copy(rows[b], acc.at[didx.at[i]], ssem[b]).start(add=True)

        def s_wait(b):
            pltpu.make_async_copy(rows[b], acc.at[didx.at[0]], ssem[b]).wait()

        with jax.named_scope("stage_zero"):
            pltpu.make_async_copy(src_ref.at[w], sidx, gsem0).start()
            pltpu.make_async_copy(dst_ref.at[w], didx, gsem1).start()
            pltpu.make_async_copy(
                z_ref, acc.at[pl.ds(s * ROWS_PER_TILE, ROWS_PER_TILE)], ssem0).start()

            pltpu.make_async_copy(src_ref.at[w], sidx, gsem0).wait()
            pltpu.make_async_copy(dst_ref.at[w], didx, gsem1).wait()
            pltpu.make_async_copy(
                z_ref, acc.at[pl.ds(s * ROWS_PER_TILE, ROWS_PER_TILE)], ssem0).wait()
            plsc.subcore_barrier()

        with jax.named_scope("edges"):
            g_start(0, 0)
            g_wait(0)
            s_start(0, 0)
            g_start(1, 1)

            def step(t, carry):
                i1 = 2 * t + 1
                g_wait(1)
                s_start(i1, 1)
                s_wait(0)
                g_start(i1 + 1, 0)
                i2 = 2 * t + 2
                g_wait(0)
                s_start(i2, 0)
                s_wait(1)
                g_start(i2 + 1, 1)
                return carry
            my_chunks = jnp.where(c == 0, CH_F, CH_S)
            lax.fori_loop(0, (my_chunks - 2) // 2, step, 0)

            g_wait(1)
            s_start(my_chunks - 1, 1)
            s_wait(0)
            s_wait(1)
            plsc.subcore_barrier()

        with jax.named_scope("dump"):
            pltpu.sync_copy(
                acc.at[pl.ds(s * ROWS_PER_TILE, ROWS_PER_TILE)],
                out_ref.at[c, pl.ds(s * ROWS_PER_TILE, ROWS_PER_TILE)],
            )

    return body(x_aug, src, dst, zrows)


R = 400


def _tc_finish(x, psum, W_self, W_neigh, bias, gamma, beta):
    def body(x_ref, p_ref, ws_ref, wn_ref, b_ref, g_ref, be_ref, o_ref):
        p = p_ref[...]
        ssum = p[0] + p[1]
        agg = ssum[:, :D]
        deg = jnp.maximum(ssum[:, D], 1.0)
        neigh = agg / deg[:, None]
        xv = x_ref[...]
        dn = (((1,), (1,)), ((), ()))
        out = (lax.dot_general(xv, ws_ref[...], dn, preferred_element_type=jnp.float32)
               + lax.dot_general(neigh, wn_ref[...], dn, preferred_element_type=jnp.float32)
               + b_ref[...])
        out = jnp.maximum(out, 0.0)
        mu = jnp.mean(out, axis=-1, keepdims=True)
        var = jnp.mean((out - mu) ** 2, axis=-1, keepdims=True)
        o_ref[...] = ((out - mu) * lax.rsqrt(var + 1e-5)) * g_ref[...] + be_ref[...]

    return pl.pallas_call(
        body,
        grid=(N // R,),
        in_specs=[
            pl.BlockSpec((R, D), lambda i: (i, 0)),
            pl.BlockSpec((NC, R, DA), lambda i: (0, i, 0)),
            pl.BlockSpec((D, D), lambda i: (0, 0)),
            pl.BlockSpec((D, D), lambda i: (0, 0)),
            pl.BlockSpec((1, D), lambda i: (0, 0)),
            pl.BlockSpec((1, D), lambda i: (0, 0)),
            pl.BlockSpec((1, D), lambda i: (0, 0)),
        ],
        out_specs=pl.BlockSpec((R, D), lambda i: (i, 0)),
        out_shape=jax.ShapeDtypeStruct((N, D), jnp.float32),
    )(x, psum, W_self, W_neigh, bias, gamma, beta)


def _split_edges(a):
    ef = NS * CH_F * K
    fast = a[:ef].reshape(NS, CH_F, K)
    slow = a[ef:].reshape(NS, CH_S, K)
    slow = jnp.pad(slow, ((0, 0), (0, CH_F - CH_S), (0, 0)))
    return jnp.stack([fast, slow], axis=1).reshape(NW, CHUNKS, K)


def kernel(x, edge_index, W_self, W_neigh, bias, ln_gamma, ln_beta):
    src = _split_edges(edge_index[0].astype(jnp.int32))
    dst = _split_edges(edge_index[1].astype(jnp.int32))
    x_aug = jnp.zeros((NPAD, DA), jnp.float32)
    x_aug = x_aug.at[:N, :D].set(x).at[:N, D].set(1.0)
    zrows = jnp.zeros((ROWS_PER_TILE, DA), jnp.float32)
    psum = _sc_aggregate(x_aug, src, dst, zrows)
    return _tc_finish(
        x, psum, W_self, W_neigh,
        bias.reshape(1, D), ln_gamma.reshape(1, D), ln_beta.reshape(1, D),
    )


# speedup vs baseline: 2.0639x; 1.2989x over previous
import functools
import jax
import jax.numpy as jnp
from jax import lax
from jax.experimental import pallas as pl
from jax.experimental.pallas import tpu as pltpu
from jax.experimental.pallas import tpu_sc as plsc

N = 10000
E = 320000
D = 128
DG = 16
NC, NS = 2, 16
NW = NC * NS
K = 40
CHUNKS = 250
NACC = 10112
ROWS_PER_TILE = NACC // NS


def _sc_aggregate(x, src, dst, zrows, zdeg):
    mesh = plsc.VectorSubcoreMesh(core_axis_name="c", subcore_axis_name="s")

    @functools.partial(
        pl.kernel,
        out_type=(
            jax.ShapeDtypeStruct((NC, NACC, D), jnp.float32),
            jax.ShapeDtypeStruct((NC, NACC, DG), jnp.float32),
        ),
        mesh=mesh,
        scratch_types=[
            pltpu.VMEM((CHUNKS, K), jnp.int32),
            pltpu.VMEM((CHUNKS, K), jnp.int32),
            pltpu.VMEM((K, D), jnp.float32),
            pltpu.VMEM((K, D), jnp.float32),
            pltpu.VMEM((K, DG), jnp.float32),
            pltpu.VMEM_SHARED((NACC, D), jnp.float32),
            pltpu.VMEM_SHARED((NACC, DG), jnp.float32),
            pltpu.SemaphoreType.DMA,
            pltpu.SemaphoreType.DMA,
            pltpu.SemaphoreType.DMA,
            pltpu.SemaphoreType.DMA,
        ],
        compiler_params=pltpu.CompilerParams(use_tc_tiling_on_sc=False),
    )
    def body(x_ref, src_ref, dst_ref, zr_ref, zd_ref, out_ref, deg_ref,
             sidx, didx, rows0, rows1, ones, acc, dacc,
             gsem0, gsem1, ssem0, ssem1):
        c = lax.axis_index("c")
        s = lax.axis_index("s")
        w = s * NC + c

        rows = (rows0, rows1)
        gsem = (gsem0, gsem1)
        ssem = (ssem0, ssem1)

        def g_start(i, b):
            pltpu.make_async_copy(x_ref.at[sidx.at[i]], rows[b], gsem[b]).start()

        def g_wait(b):
            pltpu.make_async_copy(x_ref.at[sidx.at[0]], rows[b], gsem[b]).wait()

        def s_start(i, b):
            pltpu.make_async_copy(rows[b], acc.at[didx.at[i]], ssem[b]).start(add=True)
            pltpu.make_async_copy(ones, dacc.at[didx.at[i]], ssem[b]).start(add=True)

        def s_wait(b):
            pltpu.make_async_copy(rows[b], acc.at[didx.at[0]], ssem[b]).wait()
            pltpu.make_async_copy(ones, dacc.at[didx.at[0]], ssem[b]).wait()

        stripe = pl.ds(s * ROWS_PER_TILE, ROWS_PER_TILE)

        with jax.named_scope("stage_zero"):
            pltpu.make_async_copy(src_ref.at[w], sidx, gsem0).start()
            pltpu.make_async_copy(dst_ref.at[w], didx, gsem1).start()
            pltpu.make_async_copy(zr_ref, acc.at[stripe], ssem0).start()
            pltpu.make_async_copy(zd_ref, dacc.at[stripe], ssem1).start()

            one = jnp.ones((16,), jnp.float32)
            for i in range(K):
                ones[i, :] = one

            pltpu.make_async_copy(src_ref.at[w], sidx, gsem0).wait()
            pltpu.make_async_copy(dst_ref.at[w], didx, gsem1).wait()
            pltpu.make_async_copy(zr_ref, acc.at[stripe], ssem0).wait()
            pltpu.make_async_copy(zd_ref, dacc.at[stripe], ssem1).wait()
            plsc.subcore_barrier()

        with jax.named_scope("edges"):
            g_start(0, 0)
            g_wait(0)
            s_start(0, 0)
            g_start(1, 1)

            def step(t, carry):
                i1 = 2 * t + 1
                g_wait(1)
                s_start(i1, 1)
                s_wait(0)
                g_start(i1 + 1, 0)
                i2 = 2 * t + 2
                g_wait(0)
                s_start(i2, 0)
                s_wait(1)
                g_start(i2 + 1, 1)
                return carry
            lax.fori_loop(0, (CHUNKS - 2) // 2, step, 0)

            g_wait(1)
            s_start(CHUNKS - 1, 1)
            s_wait(0)
            s_wait(1)
            plsc.subcore_barrier()

        with jax.named_scope("dump"):
            pltpu.sync_copy(acc.at[stripe], out_ref.at[c, stripe])
            pltpu.sync_copy(dacc.at[stripe], deg_ref.at[c, stripe])

    return body(x, src, dst, zrows, zdeg)


R = 400


def _tc_finish(x, psum, dsum, W_self, W_neigh, bias, gamma, beta):
    def body(x_ref, p_ref, d_ref, ws_ref, wn_ref, b_ref, g_ref, be_ref, o_ref):
        p = p_ref[...]
        agg = p[0] + p[1]
        dg = d_ref[...]
        deg = jnp.maximum(dg[0, :, 0] + dg[1, :, 0], 1.0)
        neigh = agg / deg[:, None]
        xv = x_ref[...]
        dn = (((1,), (1,)), ((), ()))
        out = (lax.dot_general(xv, ws_ref[...], dn, preferred_element_type=jnp.float32)
               + lax.dot_general(neigh, wn_ref[...], dn, preferred_element_type=jnp.float32)
               + b_ref[...])
        out = jnp.maximum(out, 0.0)
        mu = jnp.mean(out, axis=-1, keepdims=True)
        var = jnp.mean((out - mu) ** 2, axis=-1, keepdims=True)
        o_ref[...] = ((out - mu) * lax.rsqrt(var + 1e-5)) * g_ref[...] + be_ref[...]

    return pl.pallas_call(
        body,
        grid=(N // R,),
        in_specs=[
            pl.BlockSpec((R, D), lambda i: (i, 0)),
            pl.BlockSpec((NC, R, D), lambda i: (0, i, 0)),
            pl.BlockSpec((NC, R, DG), lambda i: (0, i, 0)),
            pl.BlockSpec((D, D), lambda i: (0, 0)),
            pl.BlockSpec((D, D), lambda i: (0, 0)),
            pl.BlockSpec((1, D), lambda i: (0, 0)),
            pl.BlockSpec((1, D), lambda i: (0, 0)),
            pl.BlockSpec((1, D), lambda i: (0, 0)),
        ],
        out_specs=pl.BlockSpec((R, D), lambda i: (i, 0)),
        out_shape=jax.ShapeDtypeStruct((N, D), jnp.float32),
    )(x, psum, dsum, W_self, W_neigh, bias, gamma, beta)


def kernel(x, edge_index, W_self, W_neigh, bias, ln_gamma, ln_beta):
    src = edge_index[0].astype(jnp.int32).reshape(NW, CHUNKS, K)
    dst = edge_index[1].astype(jnp.int32).reshape(NW, CHUNKS, K)
    zrows = jnp.zeros((ROWS_PER_TILE, D), jnp.float32)
    zdeg = jnp.zeros((ROWS_PER_TILE, DG), jnp.float32)
    psum, dsum = _sc_aggregate(x, src, dst, zrows, zdeg)
    return _tc_finish(
        x, psum, dsum, W_self, W_neigh,
        bias.reshape(1, D), ln_gamma.reshape(1, D), ln_beta.reshape(1, D),
    )


# speedup vs baseline: 3.3068x; 1.6022x over previous
import functools
import jax
import jax.numpy as jnp
from jax import lax
from jax.experimental import pallas as pl
from jax.experimental.pallas import tpu as pltpu
from jax.experimental.pallas import tpu_sc as plsc

N = 10000
E = 320000
D = 128
DG = 16
NC, NS = 2, 16
NW = NC * NS
K = 40
CHUNKS = 250
NACC = 10112
ROWS_PER_TILE = NACC // NS


def _sc_aggregate(x, edges, zrows, zdeg):
    mesh = plsc.VectorSubcoreMesh(core_axis_name="c", subcore_axis_name="s")

    @functools.partial(
        pl.kernel,
        out_type=(
            jax.ShapeDtypeStruct((NC, NACC, D), jnp.float32),
            jax.ShapeDtypeStruct((NC, NACC, DG), jnp.float32),
        ),
        mesh=mesh,
        scratch_types=[
            pltpu.VMEM((CHUNKS, K), jnp.int32),
            pltpu.VMEM((CHUNKS, K), jnp.int32),
            pltpu.VMEM((K, D), jnp.float32),
            pltpu.VMEM((K, D), jnp.float32),
            pltpu.VMEM((K, D), jnp.float32),
            pltpu.VMEM((K, DG), jnp.float32),
            pltpu.VMEM_SHARED((NACC, D), jnp.float32),
            pltpu.VMEM_SHARED((NACC, DG), jnp.float32),
            pltpu.SemaphoreType.DMA,
            pltpu.SemaphoreType.DMA,
            pltpu.SemaphoreType.DMA,
            pltpu.SemaphoreType.DMA,
            pltpu.SemaphoreType.DMA,
            pltpu.SemaphoreType.DMA,
            pltpu.SemaphoreType.DMA,
        ],
        compiler_params=pltpu.CompilerParams(use_tc_tiling_on_sc=False),
    )
    def body(x_ref, e_ref, zr_ref, zd_ref, out_ref, deg_ref,
             sidx, didx, rows0, rows1, rows2, ones, acc, dacc,
             gsem0, gsem1, gsem2, ssem0, ssem1, ssem2, xsem):
        c = lax.axis_index("c")
        s = lax.axis_index("s")
        w = s * NC + c

        rows = (rows0, rows1, rows2)
        gsem = (gsem0, gsem1, gsem2)
        ssem = (ssem0, ssem1, ssem2)

        def g_start(i, b):
            pltpu.make_async_copy(x_ref.at[sidx.at[i]], rows[b], gsem[b]).start()

        def g_wait(b):
            pltpu.make_async_copy(x_ref.at[sidx.at[0]], rows[b], gsem[b]).wait()

        def s_start(i, b):
            pltpu.make_async_copy(rows[b], acc.at[didx.at[i]], ssem[b]).start(add=True)
            pltpu.make_async_copy(ones, dacc.at[didx.at[i]], ssem[b]).start(add=True)

        def s_wait(b):
            pltpu.make_async_copy(rows[b], acc.at[didx.at[0]], ssem[b]).wait()
            pltpu.make_async_copy(ones, dacc.at[didx.at[0]], ssem[b]).wait()

        stripe = pl.ds(s * ROWS_PER_TILE, ROWS_PER_TILE)

        with jax.named_scope("stage_zero"):
            pltpu.make_async_copy(e_ref.at[0, w], sidx, xsem).start()
            pltpu.make_async_copy(e_ref.at[1, w], didx, xsem).start()
            pltpu.make_async_copy(zr_ref, acc.at[stripe], ssem0).start()
            pltpu.make_async_copy(zd_ref, dacc.at[stripe], ssem1).start()

            one = jnp.ones((16,), jnp.float32)
            for i in range(K):
                ones[i, :] = one

            pltpu.make_async_copy(e_ref.at[0, w], sidx, xsem).wait()
            pltpu.make_async_copy(e_ref.at[1, w], didx, xsem).wait()

        with jax.named_scope("edges"):
            g_start(0, 0)
            g_start(1, 1)
            pltpu.make_async_copy(zr_ref, acc.at[stripe], ssem0).wait()
            pltpu.make_async_copy(zd_ref, dacc.at[stripe], ssem1).wait()
            plsc.subcore_barrier()

            g_wait(0)
            s_start(0, 0)
            g_start(2, 2)
            g_wait(1)
            s_wait(0)
            s_start(1, 1)
            g_start(3, 0)

            def pos(i, b):
                g_wait(b)
                s_wait((b + 2) % 3)
                s_start(i, b)
                g_start(i + 2, (b + 2) % 3)

            def step(t, carry):
                i = 3 * t + 2
                pos(i, 2)
                pos(i + 1, 0)
                pos(i + 2, 1)
                return carry
            lax.fori_loop(0, (CHUNKS - 4) // 3, step, 0)

            g_wait(2)
            s_wait(1)
            s_start(CHUNKS - 2, 2)
            g_wait(0)
            s_wait(2)
            s_start(CHUNKS - 1, 0)
            s_wait(0)
            plsc.subcore_barrier()

        with jax.named_scope("dump"):
            pltpu.sync_copy(acc.at[stripe], out_ref.at[c, stripe])
            pltpu.sync_copy(dacc.at[stripe], deg_ref.at[c, stripe])

    return body(x, edges, zrows, zdeg)


R = 400


def _tc_finish(x, psum, dsum, W_self, W_neigh, bias, gamma, beta):
    def body(x_ref, p_ref, d_ref, ws_ref, wn_ref, b_ref, g_ref, be_ref, o_ref):
        p = p_ref[...]
        agg = p[0] + p[1]
        dg = d_ref[...]
        deg = jnp.maximum(dg[0, :, 0] + dg[1, :, 0], 1.0)
        neigh = agg / deg[:, None]
        xv = x_ref[...]
        dn = (((1,), (1,)), ((), ()))
        out = (lax.dot_general(xv, ws_ref[...], dn, preferred_element_type=jnp.float32)
               + lax.dot_general(neigh, wn_ref[...], dn, preferred_element_type=jnp.float32)
               + b_ref[...])
        out = jnp.maximum(out, 0.0)
        mu = jnp.mean(out, axis=-1, keepdims=True)
        var = jnp.mean((out - mu) ** 2, axis=-1, keepdims=True)
        o_ref[...] = ((out - mu) * lax.rsqrt(var + 1e-5)) * g_ref[...] + be_ref[...]

    return pl.pallas_call(
        body,
        grid=(N // R,),
        in_specs=[
            pl.BlockSpec((R, D), lambda i: (i, 0)),
            pl.BlockSpec((NC, R, D), lambda i: (0, i, 0)),
            pl.BlockSpec((NC, R, DG), lambda i: (0, i, 0)),
            pl.BlockSpec((D, D), lambda i: (0, 0)),
            pl.BlockSpec((D, D), lambda i: (0, 0)),
            pl.BlockSpec((1, D), lambda i: (0, 0)),
            pl.BlockSpec((1, D), lambda i: (0, 0)),
            pl.BlockSpec((1, D), lambda i: (0, 0)),
        ],
        out_specs=pl.BlockSpec((R, D), lambda i: (i, 0)),
        out_shape=jax.ShapeDtypeStruct((N, D), jnp.float32),
    )(x, psum, dsum, W_self, W_neigh, bias, gamma, beta)


def kernel(x, edge_index, W_self, W_neigh, bias, ln_gamma, ln_beta):
    edges = edge_index.astype(jnp.int32).reshape(2, NW, CHUNKS, K)
    zrows = jnp.zeros((ROWS_PER_TILE, D), jnp.float32)
    zdeg = jnp.zeros((ROWS_PER_TILE, DG), jnp.float32)
    psum, dsum = _sc_aggregate(x, edges, zrows, zdeg)
    return _tc_finish(
        x, psum, dsum, W_self, W_neigh,
        bias.reshape(1, D), ln_gamma.reshape(1, D), ln_beta.reshape(1, D),
    )
